# baseline (speedup 1.0000x reference)
import functools

import jax
import jax.numpy as jnp
from jax import lax
from jax.experimental import pallas as pl
from jax.experimental.pallas import tpu as pltpu
from jax.experimental.pallas import tpu_sc as plsc

N = 10000
E = 320000
D = 128
H = 128
O = 32
Q = 1024
MLP_H = 64

NPAD = 10240
NC = 2
NS = 16
NW = NC * NS
CHUNK = 128
EPAD = 327680
NCHUNKS = EPAD // CHUNK
CH_PER_TILE = NCHUNKS // NW
ROWS_PER_TILE = NPAD // NS

_MESH = dict(core_axis_name="c", subcore_axis_name="s", num_cores=NC,
             num_subcores=NS)
_SC_PARAMS = pltpu.CompilerParams(use_tc_tiling_on_sc=False)


def _fill_zeros_2d(ref, nrows, ncols):
    zv = jnp.zeros((16,), jnp.float32)

    def body(r, _):
        for j in range(ncols // 16):
            ref[r, pl.ds(j * 16, 16)] = zv
        return 0

    lax.fori_loop(0, nrows, body, 0)


def _make_segsum(F, K):
    mesh = plsc.VectorSubcoreMesh(**_MESH)

    @functools.partial(
        pl.kernel,
        out_type=jax.ShapeDtypeStruct((NC, NPAD, F), jnp.float32),
        mesh=mesh,
        compiler_params=_SC_PARAMS,
        scratch_types=[
            pltpu.VMEM((K, CHUNK), jnp.int32),
            pltpu.VMEM((K, CHUNK), jnp.int32),
        ] + [pltpu.VMEM((CHUNK, F), jnp.float32) for _ in range(K)] + [
            pltpu.VMEM_SHARED((NPAD, F), jnp.float32),
            pltpu.SemaphoreType.DMA,
            pltpu.SemaphoreType.DMA,
        ],
    )
    def segsum(hs, src2, dst2, out, src_blk, dst_blk, *rest):
        bufs = rest[:K]
        acc, gsem, ssem = rest[K], rest[K + 1], rest[K + 2]
        c = lax.axis_index("c")
        s = lax.axis_index("s")
        w = c * NS + s
        _fill_zeros_2d(bufs[0], CHUNK, F)
        for t in range(ROWS_PER_TILE // CHUNK):
            pltpu.sync_copy(bufs[0],
                            acc.at[pl.ds(s * ROWS_PER_TILE + t * CHUNK, CHUNK)])
        plsc.subcore_barrier()

        def blk_body(blk, _):
            row0 = w * CH_PER_TILE + blk * K
            pltpu.sync_copy(src2.at[pl.ds(row0, K), :], src_blk)
            pltpu.sync_copy(dst2.at[pl.ds(row0, K), :], dst_blk)
            g = [pltpu.async_copy(hs.at[src_blk.at[j]], bufs[j], gsem)
                 for j in range(K)]
            sc = []
            for j in range(K):
                g[j].wait()
                sc.append(pltpu.async_copy(bufs[j], acc.at[dst_blk.at[j]],
                                           ssem, add=True))
            for d in sc:
                d.wait()
            return 0

        lax.fori_loop(0, CH_PER_TILE // K, blk_body, 0)
        plsc.subcore_barrier()
        pltpu.sync_copy(acc.at[pl.ds(s * ROWS_PER_TILE, ROWS_PER_TILE)],
                        out.at[c, pl.ds(s * ROWS_PER_TILE, ROWS_PER_TILE), :])

    return segsum


_segsum_h = _make_segsum(H, 2)
_segsum_o = _make_segsum(O, 8)


_KD = 16


@functools.partial(
    pl.kernel,
    out_type=jax.ShapeDtypeStruct((NC, NPAD), jnp.float32),
    mesh=plsc.VectorSubcoreMesh(**_MESH),
    compiler_params=_SC_PARAMS,
    scratch_types=[
        pltpu.VMEM((_KD, CHUNK), jnp.int32),
        pltpu.VMEM((CHUNK,), jnp.float32),
        pltpu.VMEM((ROWS_PER_TILE,), jnp.float32),
        pltpu.VMEM_SHARED((NPAD,), jnp.float32),
        pltpu.SemaphoreType.DMA,
    ],
)
def _degree(dst2, out, dst_blk, ones_v, zer_v, acc, ssem):
    c = lax.axis_index("c")
    s = lax.axis_index("s")
    w = c * NS + s
    one = jnp.full((16,), 1.0, jnp.float32)
    zero = jnp.zeros((16,), jnp.float32)
    for j in range(CHUNK // 16):
        ones_v[pl.ds(j * 16, 16)] = one

    def zbody(r, _):
        zer_v[pl.ds(r * 16, 16)] = zero
        return 0

    lax.fori_loop(0, ROWS_PER_TILE // 16, zbody, 0)
    pltpu.sync_copy(zer_v, acc.at[pl.ds(s * ROWS_PER_TILE, ROWS_PER_TILE)])
    plsc.subcore_barrier()

    def body(blk, _):
        row0 = w * CH_PER_TILE + blk * _KD
        pltpu.sync_copy(dst2.at[pl.ds(row0, _KD), :], dst_blk)
        sc = [pltpu.async_copy(ones_v, acc.at[dst_blk.at[j]], ssem, add=True)
              for j in range(_KD)]
        for d in sc:
            d.wait()
        return 0

    lax.fori_loop(0, CH_PER_TILE // _KD, body, 0)
    plsc.subcore_barrier()
    pltpu.sync_copy(acc.at[pl.ds(s * ROWS_PER_TILE, ROWS_PER_TILE)],
                    out.at[c, pl.ds(s * ROWS_PER_TILE, ROWS_PER_TILE)])


_Q_PER_TILE = Q // (NC * NS)


@functools.partial(
    pl.kernel,
    out_type=jax.ShapeDtypeStruct((Q, O), jnp.float32),
    mesh=plsc.VectorSubcoreMesh(**_MESH),
    compiler_params=_SC_PARAMS,
    scratch_types=[
        pltpu.VMEM((_Q_PER_TILE,), jnp.int32),
        pltpu.VMEM((_Q_PER_TILE, O), jnp.float32),
        pltpu.SemaphoreType.DMA,
    ],
)
def _gather_queries(table, qidx, out, idx_v, rows_v, sem):
    c = lax.axis_index("c")
    s = lax.axis_index("s")
    w = s * NC + c
    base = w * _Q_PER_TILE
    pltpu.sync_copy(qidx.at[pl.ds(base, _Q_PER_TILE)], idx_v)
    pltpu.async_copy(table.at[idx_v], rows_v, sem).wait()
    pltpu.sync_copy(rows_v, out.at[pl.ds(base, _Q_PER_TILE), :])


_NBLK = NPAD // 1024


def _tc_scale1(x_r, w_r, dp_r, hs_r, dinv_r):
    deg = dp_r[0] + dp_r[1] + 1.0
    dinv = lax.rsqrt(jnp.maximum(deg, 1e-12))
    hs_r[...] = jnp.dot(x_r[...], w_r[...],
                        preferred_element_type=jnp.float32) * dinv
    dinv_r[...] = dinv


_stage1 = pl.pallas_call(
    _tc_scale1,
    grid=(_NBLK,),
    in_specs=[
        pl.BlockSpec((1024, D), lambda i: (i, 0)),
        pl.BlockSpec((D, H), lambda i: (0, 0)),
        pl.BlockSpec((NC, 1024, 1), lambda i: (0, i, 0)),
    ],
    out_specs=[
        pl.BlockSpec((1024, H), lambda i: (i, 0)),
        pl.BlockSpec((1024, 1), lambda i: (i, 0)),
    ],
    out_shape=[
        jax.ShapeDtypeStruct((NPAD, H), jnp.float32),
        jax.ShapeDtypeStruct((NPAD, 1), jnp.float32),
    ],
)


def _tc_combine1(p_r, hs_r, dinv_r, b1_r, w2_r, h2s_r):
    seg = p_r[0] + p_r[1] + hs_r[...]
    out1 = jnp.maximum(dinv_r[...] * seg + b1_r[...], 0.0)
    h2s_r[...] = jnp.dot(out1, w2_r[...],
                         preferred_element_type=jnp.float32) * dinv_r[...]


_stage2 = pl.pallas_call(
    _tc_combine1,
    grid=(_NBLK,),
    in_specs=[
        pl.BlockSpec((NC, 1024, H), lambda i: (0, i, 0)),
        pl.BlockSpec((1024, H), lambda i: (i, 0)),
        pl.BlockSpec((1024, 1), lambda i: (i, 0)),
        pl.BlockSpec((1, H), lambda i: (0, 0)),
        pl.BlockSpec((H, O), lambda i: (0, 0)),
    ],
    out_specs=pl.BlockSpec((1024, O), lambda i: (i, 0)),
    out_shape=jax.ShapeDtypeStruct((NPAD, O), jnp.float32),
)


def _tc_combine2(q_r, h2s_r, dinv_r, b2_r, out_r):
    seg = q_r[0] + q_r[1] + h2s_r[...]
    out_r[...] = jnp.maximum(dinv_r[...] * seg + b2_r[...], 0.0)


_stage3 = pl.pallas_call(
    _tc_combine2,
    grid=(_NBLK,),
    in_specs=[
        pl.BlockSpec((NC, 1024, O), lambda i: (0, i, 0)),
        pl.BlockSpec((1024, O), lambda i: (i, 0)),
        pl.BlockSpec((1024, 1), lambda i: (i, 0)),
        pl.BlockSpec((1, O), lambda i: (0, 0)),
    ],
    out_specs=pl.BlockSpec((1024, O), lambda i: (i, 0)),
    out_shape=jax.ShapeDtypeStruct((NPAD, O), jnp.float32),
)


def _tc_mlp(q_r, wm1_r, bm1_r, wm2_r, bm2_r, out_r):
    z = jnp.maximum(jnp.dot(q_r[...], wm1_r[...],
                            preferred_element_type=jnp.float32) + bm1_r[...], 0.0)
    out_r[...] = jnp.dot(z, wm2_r[...],
                         preferred_element_type=jnp.float32) + bm2_r[...]


_mlp = pl.pallas_call(
    _tc_mlp,
    out_shape=jax.ShapeDtypeStruct((Q, 1), jnp.float32),
)


@jax.jit
def kernel(x, edge_index, query_node_indices, W1, b1, W2, b2, Wm1, bm1, Wm2, bm2):
    npad_e = EPAD - E
    src2 = jnp.concatenate(
        [edge_index[0], jnp.zeros((npad_e,), jnp.int32)]).reshape(NCHUNKS, CHUNK)
    dst2 = jnp.concatenate(
        [edge_index[1], jnp.full((npad_e,), N, jnp.int32)]).reshape(NCHUNKS, CHUNK)
    x_pad = jnp.zeros((NPAD, D), jnp.float32).at[:N].set(x)

    degp = _degree(dst2)
    hs, dinv = _stage1(x_pad, W1, degp.reshape(NC, NPAD, 1))
    p1 = _segsum_h(hs, src2, dst2)
    h2s = _stage2(p1, hs, dinv, b1.reshape(1, H), W2)
    p2 = _segsum_o(h2s, src2, dst2)
    out2 = _stage3(p2, h2s, dinv, b2.reshape(1, O))
    qrows = _gather_queries(out2, query_node_indices)
    logits = _mlp(qrows, Wm1, bm1.reshape(1, MLP_H), Wm2, bm2.reshape(1, 1))
    return logits[:, 0]


# speedup vs baseline: 1.0062x; 1.0062x over previous
import functools

import jax
import jax.numpy as jnp
from jax import lax
from jax.experimental import pallas as pl
from jax.experimental.pallas import tpu as pltpu
from jax.experimental.pallas import tpu_sc as plsc

N = 10000
E = 320000
D = 128
H = 128
O = 32
Q = 1024
MLP_H = 64

NPAD = 10240
NC = 2
NS = 16
NW = NC * NS
CHUNK = 128
EPAD = 327680
NCHUNKS = EPAD // CHUNK
CH_PER_TILE = NCHUNKS // NW
ROWS_PER_TILE = NPAD // NS

_MESH = dict(core_axis_name="c", subcore_axis_name="s", num_cores=NC,
             num_subcores=NS)
_SC_PARAMS = pltpu.CompilerParams(use_tc_tiling_on_sc=False)


def _fill_zeros_2d(ref, nrows, ncols):
    zv = jnp.zeros((16,), jnp.float32)

    def body(r, _):
        for j in range(ncols // 16):
            ref[r, pl.ds(j * 16, 16)] = zv
        return 0

    lax.fori_loop(0, nrows, body, 0)


def _make_segsum(F, K):
    mesh = plsc.VectorSubcoreMesh(**_MESH)

    @functools.partial(
        pl.kernel,
        out_type=jax.ShapeDtypeStruct((NC, NPAD, F), jnp.float32),
        mesh=mesh,
        compiler_params=_SC_PARAMS,
        scratch_types=[
            pltpu.VMEM((K, CHUNK), jnp.int32),
            pltpu.VMEM((K, CHUNK), jnp.int32),
        ] + [pltpu.VMEM((CHUNK, F), jnp.float32) for _ in range(K)] + [
            pltpu.VMEM_SHARED((NPAD, F), jnp.float32),
            pltpu.SemaphoreType.DMA,
            pltpu.SemaphoreType.DMA,
        ],
    )
    def segsum(hs, src2, dst2, out, src_blk, dst_blk, *rest):
        bufs = rest[:K]
        acc, gsem, ssem = rest[K], rest[K + 1], rest[K + 2]
        c = lax.axis_index("c")
        s = lax.axis_index("s")
        w = c * NS + s
        _fill_zeros_2d(bufs[0], CHUNK, F)
        for t in range(ROWS_PER_TILE // CHUNK):
            pltpu.sync_copy(bufs[0],
                            acc.at[pl.ds(s * ROWS_PER_TILE + t * CHUNK, CHUNK)])
        plsc.subcore_barrier()

        def blk_body(blk, _):
            row0 = w * CH_PER_TILE + blk * K
            pltpu.sync_copy(src2.at[pl.ds(row0, K), :], src_blk)
            pltpu.sync_copy(dst2.at[pl.ds(row0, K), :], dst_blk)
            g = [pltpu.async_copy(hs.at[src_blk.at[j]], bufs[j], gsem)
                 for j in range(K)]
            sc = []
            for j in range(K):
                g[j].wait()
                sc.append(pltpu.async_copy(bufs[j], acc.at[dst_blk.at[j]],
                                           ssem, add=True))
            for d in sc:
                d.wait()
            return 0

        lax.fori_loop(0, CH_PER_TILE // K, blk_body, 0)
        plsc.subcore_barrier()
        pltpu.sync_copy(acc.at[pl.ds(s * ROWS_PER_TILE, ROWS_PER_TILE)],
                        out.at[c, pl.ds(s * ROWS_PER_TILE, ROWS_PER_TILE), :])

    return segsum


_segsum_h = _make_segsum(H, 2)
_segsum_o = _make_segsum(O, 8)


_KD = 16


@functools.partial(
    pl.kernel,
    out_type=jax.ShapeDtypeStruct((NC, NPAD), jnp.float32),
    mesh=plsc.VectorSubcoreMesh(**_MESH),
    compiler_params=_SC_PARAMS,
    scratch_types=[
        pltpu.VMEM((_KD, CHUNK), jnp.int32),
        pltpu.VMEM((CHUNK,), jnp.float32),
        pltpu.VMEM((ROWS_PER_TILE,), jnp.float32),
        pltpu.VMEM_SHARED((NPAD,), jnp.float32),
        pltpu.SemaphoreType.DMA,
    ],
)
def _degree(dst2, out, dst_blk, ones_v, zer_v, acc, ssem):
    c = lax.axis_index("c")
    s = lax.axis_index("s")
    w = c * NS + s
    one = jnp.full((16,), 1.0, jnp.float32)
    zero = jnp.zeros((16,), jnp.float32)
    for j in range(CHUNK // 16):
        ones_v[pl.ds(j * 16, 16)] = one

    def zbody(r, _):
        zer_v[pl.ds(r * 16, 16)] = zero
        return 0

    lax.fori_loop(0, ROWS_PER_TILE // 16, zbody, 0)
    pltpu.sync_copy(zer_v, acc.at[pl.ds(s * ROWS_PER_TILE, ROWS_PER_TILE)])
    plsc.subcore_barrier()

    def body(blk, _):
        row0 = w * CH_PER_TILE + blk * _KD
        pltpu.sync_copy(dst2.at[pl.ds(row0, _KD), :], dst_blk)
        sc = [pltpu.async_copy(ones_v, acc.at[dst_blk.at[j]], ssem, add=True)
              for j in range(_KD)]
        for d in sc:
            d.wait()
        return 0

    lax.fori_loop(0, CH_PER_TILE // _KD, body, 0)
    plsc.subcore_barrier()
    pltpu.sync_copy(acc.at[pl.ds(s * ROWS_PER_TILE, ROWS_PER_TILE)],
                    out.at[c, pl.ds(s * ROWS_PER_TILE, ROWS_PER_TILE)])


_Q_PER_TILE = Q // (NC * NS)


@functools.partial(
    pl.kernel,
    out_type=jax.ShapeDtypeStruct((Q, O), jnp.float32),
    mesh=plsc.VectorSubcoreMesh(**_MESH),
    compiler_params=_SC_PARAMS,
    scratch_types=[
        pltpu.VMEM((_Q_PER_TILE,), jnp.int32),
        pltpu.VMEM((_Q_PER_TILE, O), jnp.float32),
        pltpu.SemaphoreType.DMA,
    ],
)
def _gather_queries(table, qidx, out, idx_v, rows_v, sem):
    c = lax.axis_index("c")
    s = lax.axis_index("s")
    w = s * NC + c
    base = w * _Q_PER_TILE
    pltpu.sync_copy(qidx.at[pl.ds(base, _Q_PER_TILE)], idx_v)
    pltpu.async_copy(table.at[idx_v], rows_v, sem).wait()
    pltpu.sync_copy(rows_v, out.at[pl.ds(base, _Q_PER_TILE), :])


_NBLK = NPAD // 1024


def _tc_scale1(x_r, w_r, dp_r, hs_r, dinv_r):
    deg = dp_r[0] + dp_r[1] + 1.0
    dinv = lax.rsqrt(jnp.maximum(deg, 1e-12))
    hs_r[...] = jnp.dot(x_r[...], w_r[...],
                        preferred_element_type=jnp.float32) * dinv
    dinv_r[...] = dinv


_stage1 = pl.pallas_call(
    _tc_scale1,
    grid=(_NBLK,),
    in_specs=[
        pl.BlockSpec((1024, D), lambda i: (i, 0)),
        pl.BlockSpec((D, H), lambda i: (0, 0)),
        pl.BlockSpec((NC, 1024, 1), lambda i: (0, i, 0)),
    ],
    out_specs=[
        pl.BlockSpec((1024, H), lambda i: (i, 0)),
        pl.BlockSpec((1024, 1), lambda i: (i, 0)),
    ],
    out_shape=[
        jax.ShapeDtypeStruct((NPAD, H), jnp.float32),
        jax.ShapeDtypeStruct((NPAD, 1), jnp.float32),
    ],
)


def _tc_combine1(p_r, hs_r, dinv_r, b1_r, w2_r, h2s_r):
    seg = p_r[0] + p_r[1] + hs_r[...]
    out1 = jnp.maximum(dinv_r[...] * seg + b1_r[...], 0.0)
    h2s_r[...] = jnp.dot(out1, w2_r[...],
                         preferred_element_type=jnp.float32) * dinv_r[...]


_stage2 = pl.pallas_call(
    _tc_combine1,
    grid=(_NBLK,),
    in_specs=[
        pl.BlockSpec((NC, 1024, H), lambda i: (0, i, 0)),
        pl.BlockSpec((1024, H), lambda i: (i, 0)),
        pl.BlockSpec((1024, 1), lambda i: (i, 0)),
        pl.BlockSpec((1, H), lambda i: (0, 0)),
        pl.BlockSpec((H, O), lambda i: (0, 0)),
    ],
    out_specs=pl.BlockSpec((1024, O), lambda i: (i, 0)),
    out_shape=jax.ShapeDtypeStruct((NPAD, O), jnp.float32),
)


def _tc_combine2(q_r, h2s_r, dinv_r, b2_r, out_r):
    seg = q_r[0] + q_r[1] + h2s_r[...]
    out_r[...] = jnp.maximum(dinv_r[...] * seg + b2_r[...], 0.0)


_stage3 = pl.pallas_call(
    _tc_combine2,
    grid=(_NBLK,),
    in_specs=[
        pl.BlockSpec((NC, 1024, O), lambda i: (0, i, 0)),
        pl.BlockSpec((1024, O), lambda i: (i, 0)),
        pl.BlockSpec((1024, 1), lambda i: (i, 0)),
        pl.BlockSpec((1, O), lambda i: (0, 0)),
    ],
    out_specs=pl.BlockSpec((1024, O), lambda i: (i, 0)),
    out_shape=jax.ShapeDtypeStruct((NPAD, O), jnp.float32),
)


def _tc_mlp(q_r, wm1_r, bm1_r, wm2_r, bm2_r, out_r):
    z = jnp.maximum(jnp.dot(q_r[...], wm1_r[...],
                            preferred_element_type=jnp.float32) + bm1_r[...], 0.0)
    out_r[...] = jnp.dot(z, wm2_r[...],
                         preferred_element_type=jnp.float32) + bm2_r[...]


_mlp = pl.pallas_call(
    _tc_mlp,
    out_shape=jax.ShapeDtypeStruct((Q, 1), jnp.float32),
)


@jax.jit
def kernel(x, edge_index, query_node_indices, W1, b1, W2, b2, Wm1, bm1, Wm2, bm2):
    npad_e = EPAD - E
    pad_dst = N + jnp.arange(npad_e, dtype=jnp.int32) % (NPAD - N)
    src2 = jnp.concatenate(
        [edge_index[0], jnp.zeros((npad_e,), jnp.int32)]).reshape(NCHUNKS, CHUNK)
    dst2 = jnp.concatenate(
        [edge_index[1], pad_dst]).reshape(NCHUNKS, CHUNK)
    x_pad = jnp.zeros((NPAD, D), jnp.float32).at[:N].set(x)

    degp = _degree(dst2)
    hs, dinv = _stage1(x_pad, W1, degp.reshape(NC, NPAD, 1))
    p1 = _segsum_h(hs, src2, dst2)
    h2s = _stage2(p1, hs, dinv, b1.reshape(1, H), W2)
    p2 = _segsum_o(h2s, src2, dst2)
    out2 = _stage3(p2, h2s, dinv, b2.reshape(1, O))
    qrows = _gather_queries(out2, query_node_indices)
    logits = _mlp(qrows, Wm1, bm1.reshape(1, MLP_H), Wm2, bm2.reshape(1, 1))
    return logits[:, 0]


# speedup vs baseline: 1.2792x; 1.2713x over previous
import functools

import jax
import jax.numpy as jnp
from jax import lax
from jax.experimental import pallas as pl
from jax.experimental.pallas import tpu as pltpu
from jax.experimental.pallas import tpu_sc as plsc

N = 10000
E = 320000
D = 128
H = 128
O = 32
Q = 1024
MLP_H = 64

NPAD = 10240
NC = 2
NS = 16
NW = NC * NS
CHUNK = 128
EPAD = 327680
NCHUNKS = EPAD // CHUNK
CH_PER_TILE = NCHUNKS // NW
ROWS_PER_TILE = NPAD // NS

_MESH = dict(core_axis_name="c", subcore_axis_name="s", num_cores=NC,
             num_subcores=NS)
_SC_PARAMS = pltpu.CompilerParams(use_tc_tiling_on_sc=False)


def _fill_zeros_2d(ref, nrows, ncols):
    zv = jnp.zeros((16,), jnp.float32)

    def body(r, _):
        for j in range(ncols // 16):
            ref[r, pl.ds(j * 16, 16)] = zv
        return 0

    lax.fori_loop(0, nrows, body, 0)


def _make_segsum(F, K):
    mesh = plsc.VectorSubcoreMesh(**_MESH)

    @functools.partial(
        pl.kernel,
        out_type=jax.ShapeDtypeStruct((NC, NPAD, F), jnp.float32),
        mesh=mesh,
        compiler_params=_SC_PARAMS,
        scratch_types=[
            pltpu.VMEM((K, CHUNK), jnp.int32),
            pltpu.VMEM((K, CHUNK), jnp.int32),
        ] + [pltpu.VMEM((CHUNK, F), jnp.float32) for _ in range(K)] + [
            pltpu.VMEM_SHARED((NPAD, F), jnp.float32),
            pltpu.SemaphoreType.DMA,
            pltpu.SemaphoreType.DMA,
        ],
    )
    def segsum(hs, src2, dst2, out, src_blk, dst_blk, *rest):
        bufs = rest[:K]
        acc, gsem, ssem = rest[K], rest[K + 1], rest[K + 2]
        c = lax.axis_index("c")
        s = lax.axis_index("s")
        w = c * NS + s
        _fill_zeros_2d(bufs[0], CHUNK, F)
        for t in range(ROWS_PER_TILE // CHUNK):
            pltpu.sync_copy(bufs[0],
                            acc.at[pl.ds(s * ROWS_PER_TILE + t * CHUNK, CHUNK)])
        plsc.subcore_barrier()

        def blk_body(blk, _):
            row0 = (blk * NW + w) * K
            pltpu.sync_copy(src2.at[pl.ds(row0, K), :], src_blk)
            pltpu.sync_copy(dst2.at[pl.ds(row0, K), :], dst_blk)
            g = [pltpu.async_copy(hs.at[src_blk.at[j]], bufs[j], gsem)
                 for j in range(K)]
            sc = []
            for j in range(K):
                g[j].wait()
                sc.append(pltpu.async_copy(bufs[j], acc.at[dst_blk.at[j]],
                                           ssem, add=True))
            for d in sc:
                d.wait()
            return 0

        lax.fori_loop(0, CH_PER_TILE // K, blk_body, 0)
        plsc.subcore_barrier()
        pltpu.sync_copy(acc.at[pl.ds(s * ROWS_PER_TILE, ROWS_PER_TILE)],
                        out.at[c, pl.ds(s * ROWS_PER_TILE, ROWS_PER_TILE), :])

    return segsum


_segsum_h = _make_segsum(H, 2)
_segsum_o = _make_segsum(O, 8)


_KD = 16


@functools.partial(
    pl.kernel,
    out_type=jax.ShapeDtypeStruct((NC, NPAD), jnp.float32),
    mesh=plsc.VectorSubcoreMesh(**_MESH),
    compiler_params=_SC_PARAMS,
    scratch_types=[
        pltpu.VMEM((_KD, CHUNK), jnp.int32),
        pltpu.VMEM((CHUNK,), jnp.float32),
        pltpu.VMEM((ROWS_PER_TILE,), jnp.float32),
        pltpu.VMEM_SHARED((NPAD,), jnp.float32),
        pltpu.SemaphoreType.DMA,
    ],
)
def _degree(dst2, out, dst_blk, ones_v, zer_v, acc, ssem):
    c = lax.axis_index("c")
    s = lax.axis_index("s")
    w = c * NS + s
    one = jnp.full((16,), 1.0, jnp.float32)
    zero = jnp.zeros((16,), jnp.float32)
    for j in range(CHUNK // 16):
        ones_v[pl.ds(j * 16, 16)] = one

    def zbody(r, _):
        zer_v[pl.ds(r * 16, 16)] = zero
        return 0

    lax.fori_loop(0, ROWS_PER_TILE // 16, zbody, 0)
    pltpu.sync_copy(zer_v, acc.at[pl.ds(s * ROWS_PER_TILE, ROWS_PER_TILE)])
    plsc.subcore_barrier()

    def body(blk, _):
        row0 = w * CH_PER_TILE + blk * _KD
        pltpu.sync_copy(dst2.at[pl.ds(row0, _KD), :], dst_blk)
        sc = [pltpu.async_copy(ones_v, acc.at[dst_blk.at[j]], ssem, add=True)
              for j in range(_KD)]
        for d in sc:
            d.wait()
        return 0

    lax.fori_loop(0, CH_PER_TILE // _KD, body, 0)
    plsc.subcore_barrier()
    pltpu.sync_copy(acc.at[pl.ds(s * ROWS_PER_TILE, ROWS_PER_TILE)],
                    out.at[c, pl.ds(s * ROWS_PER_TILE, ROWS_PER_TILE)])


_Q_PER_TILE = Q // (NC * NS)


@functools.partial(
    pl.kernel,
    out_type=jax.ShapeDtypeStruct((Q, O), jnp.float32),
    mesh=plsc.VectorSubcoreMesh(**_MESH),
    compiler_params=_SC_PARAMS,
    scratch_types=[
        pltpu.VMEM((_Q_PER_TILE,), jnp.int32),
        pltpu.VMEM((_Q_PER_TILE, O), jnp.float32),
        pltpu.SemaphoreType.DMA,
    ],
)
def _gather_queries(table, qidx, out, idx_v, rows_v, sem):
    c = lax.axis_index("c")
    s = lax.axis_index("s")
    w = s * NC + c
    base = w * _Q_PER_TILE
    pltpu.sync_copy(qidx.at[pl.ds(base, _Q_PER_TILE)], idx_v)
    pltpu.async_copy(table.at[idx_v], rows_v, sem).wait()
    pltpu.sync_copy(rows_v, out.at[pl.ds(base, _Q_PER_TILE), :])


_NBLK = NPAD // 1024


def _tc_scale1(x_r, w_r, dp_r, hs_r, dinv_r):
    deg = dp_r[0] + dp_r[1] + 1.0
    dinv = lax.rsqrt(jnp.maximum(deg, 1e-12))
    hs_r[...] = jnp.dot(x_r[...], w_r[...],
                        preferred_element_type=jnp.float32) * dinv
    dinv_r[...] = dinv


_stage1 = pl.pallas_call(
    _tc_scale1,
    grid=(_NBLK,),
    in_specs=[
        pl.BlockSpec((1024, D), lambda i: (i, 0)),
        pl.BlockSpec((D, H), lambda i: (0, 0)),
        pl.BlockSpec((NC, 1024, 1), lambda i: (0, i, 0)),
    ],
    out_specs=[
        pl.BlockSpec((1024, H), lambda i: (i, 0)),
        pl.BlockSpec((1024, 1), lambda i: (i, 0)),
    ],
    out_shape=[
        jax.ShapeDtypeStruct((NPAD, H), jnp.float32),
        jax.ShapeDtypeStruct((NPAD, 1), jnp.float32),
    ],
)


def _tc_combine1(p_r, hs_r, dinv_r, b1_r, w2_r, h2s_r):
    seg = p_r[0] + p_r[1] + hs_r[...]
    out1 = jnp.maximum(dinv_r[...] * seg + b1_r[...], 0.0)
    h2s_r[...] = jnp.dot(out1, w2_r[...],
                         preferred_element_type=jnp.float32) * dinv_r[...]


_stage2 = pl.pallas_call(
    _tc_combine1,
    grid=(_NBLK,),
    in_specs=[
        pl.BlockSpec((NC, 1024, H), lambda i: (0, i, 0)),
        pl.BlockSpec((1024, H), lambda i: (i, 0)),
        pl.BlockSpec((1024, 1), lambda i: (i, 0)),
        pl.BlockSpec((1, H), lambda i: (0, 0)),
        pl.BlockSpec((H, O), lambda i: (0, 0)),
    ],
    out_specs=pl.BlockSpec((1024, O), lambda i: (i, 0)),
    out_shape=jax.ShapeDtypeStruct((NPAD, O), jnp.float32),
)


def _tc_combine2(q_r, h2s_r, dinv_r, b2_r, out_r):
    seg = q_r[0] + q_r[1] + h2s_r[...]
    out_r[...] = jnp.maximum(dinv_r[...] * seg + b2_r[...], 0.0)


_stage3 = pl.pallas_call(
    _tc_combine2,
    grid=(_NBLK,),
    in_specs=[
        pl.BlockSpec((NC, 1024, O), lambda i: (0, i, 0)),
        pl.BlockSpec((1024, O), lambda i: (i, 0)),
        pl.BlockSpec((1024, 1), lambda i: (i, 0)),
        pl.BlockSpec((1, O), lambda i: (0, 0)),
    ],
    out_specs=pl.BlockSpec((1024, O), lambda i: (i, 0)),
    out_shape=jax.ShapeDtypeStruct((NPAD, O), jnp.float32),
)


def _tc_mlp(q_r, wm1_r, bm1_r, wm2_r, bm2_r, out_r):
    z = jnp.maximum(jnp.dot(q_r[...], wm1_r[...],
                            preferred_element_type=jnp.float32) + bm1_r[...], 0.0)
    out_r[...] = jnp.dot(z, wm2_r[...],
                         preferred_element_type=jnp.float32) + bm2_r[...]


_mlp = pl.pallas_call(
    _tc_mlp,
    out_shape=jax.ShapeDtypeStruct((Q, 1), jnp.float32),
)


@jax.jit
def kernel(x, edge_index, query_node_indices, W1, b1, W2, b2, Wm1, bm1, Wm2, bm2):
    npad_e = EPAD - E
    pad_dst = N + jnp.arange(npad_e, dtype=jnp.int32) % (NPAD - N)
    src2 = jnp.concatenate(
        [edge_index[0], jnp.zeros((npad_e,), jnp.int32)]).reshape(NCHUNKS, CHUNK)
    dst2 = jnp.concatenate(
        [edge_index[1], pad_dst]).reshape(NCHUNKS, CHUNK)
    x_pad = jnp.zeros((NPAD, D), jnp.float32).at[:N].set(x)

    degp = _degree(dst2)
    hs, dinv = _stage1(x_pad, W1, degp.reshape(NC, NPAD, 1))
    p1 = _segsum_h(hs, src2, dst2)
    h2s = _stage2(p1, hs, dinv, b1.reshape(1, H), W2)
    p2 = _segsum_o(h2s, src2, dst2)
    out2 = _stage3(p2, h2s, dinv, b2.reshape(1, O))
    qrows = _gather_queries(out2, query_node_indices)
    logits = _mlp(qrows, Wm1, bm1.reshape(1, MLP_H), Wm2, bm2.reshape(1, 1))
    return logits[:, 0]


# speedup vs baseline: 1.2840x; 1.0037x over previous
import functools

import jax
import jax.numpy as jnp
from jax import lax
from jax.experimental import pallas as pl
from jax.experimental.pallas import tpu as pltpu
from jax.experimental.pallas import tpu_sc as plsc

N = 10000
E = 320000
D = 128
H = 128
O = 32
Q = 1024
MLP_H = 64

NPAD = 10240
NC = 2
NS = 16
NW = NC * NS
CHUNK = 128
EPAD = 327680
NCHUNKS = EPAD // CHUNK
CH_PER_TILE = NCHUNKS // NW
ROWS_PER_TILE = NPAD // NS

_MESH = dict(core_axis_name="c", subcore_axis_name="s", num_cores=NC,
             num_subcores=NS)
_SC_PARAMS = pltpu.CompilerParams(use_tc_tiling_on_sc=False)


def _fill_zeros_2d(ref, nrows, ncols):
    zv = jnp.zeros((16,), jnp.float32)

    def body(r, _):
        for j in range(ncols // 16):
            ref[r, pl.ds(j * 16, 16)] = zv
        return 0

    lax.fori_loop(0, nrows, body, 0)


def _make_segsum(F, K):
    mesh = plsc.VectorSubcoreMesh(**_MESH)

    @functools.partial(
        pl.kernel,
        out_type=jax.ShapeDtypeStruct((NC, NPAD, F), jnp.float32),
        mesh=mesh,
        compiler_params=_SC_PARAMS,
        scratch_types=[
            pltpu.VMEM((K, CHUNK), jnp.int32),
            pltpu.VMEM((K, CHUNK), jnp.int32),
        ] + [pltpu.VMEM((CHUNK, F), jnp.float32) for _ in range(K)] + [
            pltpu.VMEM_SHARED((NPAD, F), jnp.float32),
            pltpu.SemaphoreType.DMA,
            pltpu.SemaphoreType.DMA,
        ],
    )
    def segsum(hs, src2, dst2, out, src_blk, dst_blk, *rest):
        bufs = rest[:K]
        acc, gsem, ssem = rest[K], rest[K + 1], rest[K + 2]
        c = lax.axis_index("c")
        s = lax.axis_index("s")
        w = c * NS + s
        _fill_zeros_2d(bufs[0], CHUNK, F)
        for t in range(ROWS_PER_TILE // CHUNK):
            pltpu.sync_copy(bufs[0],
                            acc.at[pl.ds(s * ROWS_PER_TILE + t * CHUNK, CHUNK)])
        plsc.subcore_barrier()

        def blk_body(blk, _):
            row0 = (blk * NW + w) * K
            pltpu.sync_copy(src2.at[pl.ds(row0, K), :], src_blk)
            pltpu.sync_copy(dst2.at[pl.ds(row0, K), :], dst_blk)
            g = [pltpu.async_copy(hs.at[src_blk.at[j]], bufs[j], gsem)
                 for j in range(K)]
            sc = []
            for j in range(K):
                g[j].wait()
                sc.append(pltpu.async_copy(bufs[j], acc.at[dst_blk.at[j]],
                                           ssem, add=True))
            for d in sc:
                d.wait()
            return 0

        lax.fori_loop(0, CH_PER_TILE // K, blk_body, 0)
        plsc.subcore_barrier()
        pltpu.sync_copy(acc.at[pl.ds(s * ROWS_PER_TILE, ROWS_PER_TILE)],
                        out.at[c, pl.ds(s * ROWS_PER_TILE, ROWS_PER_TILE), :])

    return segsum


FH = H // 2
_segsum_h = _make_segsum(FH, 8)
_segsum_o = _make_segsum(O, 8)


_KD = 16


@functools.partial(
    pl.kernel,
    out_type=jax.ShapeDtypeStruct((NC, NPAD), jnp.float32),
    mesh=plsc.VectorSubcoreMesh(**_MESH),
    compiler_params=_SC_PARAMS,
    scratch_types=[
        pltpu.VMEM((_KD, CHUNK), jnp.int32),
        pltpu.VMEM((CHUNK,), jnp.float32),
        pltpu.VMEM((ROWS_PER_TILE,), jnp.float32),
        pltpu.VMEM_SHARED((NPAD,), jnp.float32),
        pltpu.SemaphoreType.DMA,
    ],
)
def _degree(dst2, out, dst_blk, ones_v, zer_v, acc, ssem):
    c = lax.axis_index("c")
    s = lax.axis_index("s")
    w = c * NS + s
    one = jnp.full((16,), 1.0, jnp.float32)
    zero = jnp.zeros((16,), jnp.float32)
    for j in range(CHUNK // 16):
        ones_v[pl.ds(j * 16, 16)] = one

    def zbody(r, _):
        zer_v[pl.ds(r * 16, 16)] = zero
        return 0

    lax.fori_loop(0, ROWS_PER_TILE // 16, zbody, 0)
    pltpu.sync_copy(zer_v, acc.at[pl.ds(s * ROWS_PER_TILE, ROWS_PER_TILE)])
    plsc.subcore_barrier()

    def body(blk, _):
        row0 = w * CH_PER_TILE + blk * _KD
        pltpu.sync_copy(dst2.at[pl.ds(row0, _KD), :], dst_blk)
        sc = [pltpu.async_copy(ones_v, acc.at[dst_blk.at[j]], ssem, add=True)
              for j in range(_KD)]
        for d in sc:
            d.wait()
        return 0

    lax.fori_loop(0, CH_PER_TILE // _KD, body, 0)
    plsc.subcore_barrier()
    pltpu.sync_copy(acc.at[pl.ds(s * ROWS_PER_TILE, ROWS_PER_TILE)],
                    out.at[c, pl.ds(s * ROWS_PER_TILE, ROWS_PER_TILE)])


_Q_PER_TILE = Q // (NC * NS)


@functools.partial(
    pl.kernel,
    out_type=jax.ShapeDtypeStruct((Q, O), jnp.float32),
    mesh=plsc.VectorSubcoreMesh(**_MESH),
    compiler_params=_SC_PARAMS,
    scratch_types=[
        pltpu.VMEM((_Q_PER_TILE,), jnp.int32),
        pltpu.VMEM((_Q_PER_TILE, O), jnp.float32),
        pltpu.SemaphoreType.DMA,
    ],
)
def _gather_queries(table, qidx, out, idx_v, rows_v, sem):
    c = lax.axis_index("c")
    s = lax.axis_index("s")
    w = s * NC + c
    base = w * _Q_PER_TILE
    pltpu.sync_copy(qidx.at[pl.ds(base, _Q_PER_TILE)], idx_v)
    pltpu.async_copy(table.at[idx_v], rows_v, sem).wait()
    pltpu.sync_copy(rows_v, out.at[pl.ds(base, _Q_PER_TILE), :])


_NBLK = NPAD // 1024


def _tc_scale1(x_r, w_r, dp_r, hsa_r, hsb_r, dinv_r):
    deg = dp_r[0] + dp_r[1] + 1.0
    dinv = lax.rsqrt(jnp.maximum(deg, 1e-12))
    hs = jnp.dot(x_r[...], w_r[...],
                 preferred_element_type=jnp.float32) * dinv
    hsa_r[...] = hs[:, :FH]
    hsb_r[...] = hs[:, FH:]
    dinv_r[...] = dinv


_stage1 = pl.pallas_call(
    _tc_scale1,
    grid=(_NBLK,),
    in_specs=[
        pl.BlockSpec((1024, D), lambda i: (i, 0)),
        pl.BlockSpec((D, H), lambda i: (0, 0)),
        pl.BlockSpec((NC, 1024, 1), lambda i: (0, i, 0)),
    ],
    out_specs=[
        pl.BlockSpec((1024, FH), lambda i: (i, 0)),
        pl.BlockSpec((1024, FH), lambda i: (i, 0)),
        pl.BlockSpec((1024, 1), lambda i: (i, 0)),
    ],
    out_shape=[
        jax.ShapeDtypeStruct((NPAD, FH), jnp.float32),
        jax.ShapeDtypeStruct((NPAD, FH), jnp.float32),
        jax.ShapeDtypeStruct((NPAD, 1), jnp.float32),
    ],
)


def _tc_combine1(pa_r, pb_r, hsa_r, hsb_r, dinv_r, b1_r, w2_r, h2s_r):
    seg_a = pa_r[0] + pa_r[1] + hsa_r[...]
    seg_b = pb_r[0] + pb_r[1] + hsb_r[...]
    seg = jnp.concatenate([seg_a, seg_b], axis=1)
    out1 = jnp.maximum(dinv_r[...] * seg + b1_r[...], 0.0)
    h2s_r[...] = jnp.dot(out1, w2_r[...],
                         preferred_element_type=jnp.float32) * dinv_r[...]


_stage2 = pl.pallas_call(
    _tc_combine1,
    grid=(_NBLK,),
    in_specs=[
        pl.BlockSpec((NC, 1024, FH), lambda i: (0, i, 0)),
        pl.BlockSpec((NC, 1024, FH), lambda i: (0, i, 0)),
        pl.BlockSpec((1024, FH), lambda i: (i, 0)),
        pl.BlockSpec((1024, FH), lambda i: (i, 0)),
        pl.BlockSpec((1024, 1), lambda i: (i, 0)),
        pl.BlockSpec((1, H), lambda i: (0, 0)),
        pl.BlockSpec((H, O), lambda i: (0, 0)),
    ],
    out_specs=pl.BlockSpec((1024, O), lambda i: (i, 0)),
    out_shape=jax.ShapeDtypeStruct((NPAD, O), jnp.float32),
)


def _tc_combine2(q_r, h2s_r, dinv_r, b2_r, out_r):
    seg = q_r[0] + q_r[1] + h2s_r[...]
    out_r[...] = jnp.maximum(dinv_r[...] * seg + b2_r[...], 0.0)


_stage3 = pl.pallas_call(
    _tc_combine2,
    grid=(_NBLK,),
    in_specs=[
        pl.BlockSpec((NC, 1024, O), lambda i: (0, i, 0)),
        pl.BlockSpec((1024, O), lambda i: (i, 0)),
        pl.BlockSpec((1024, 1), lambda i: (i, 0)),
        pl.BlockSpec((1, O), lambda i: (0, 0)),
    ],
    out_specs=pl.BlockSpec((1024, O), lambda i: (i, 0)),
    out_shape=jax.ShapeDtypeStruct((NPAD, O), jnp.float32),
)


def _tc_mlp(q_r, wm1_r, bm1_r, wm2_r, bm2_r, out_r):
    z = jnp.maximum(jnp.dot(q_r[...], wm1_r[...],
                            preferred_element_type=jnp.float32) + bm1_r[...], 0.0)
    out_r[...] = jnp.dot(z, wm2_r[...],
                         preferred_element_type=jnp.float32) + bm2_r[...]


_mlp = pl.pallas_call(
    _tc_mlp,
    out_shape=jax.ShapeDtypeStruct((Q, 1), jnp.float32),
)


@jax.jit
def kernel(x, edge_index, query_node_indices, W1, b1, W2, b2, Wm1, bm1, Wm2, bm2):
    npad_e = EPAD - E
    pad_dst = N + jnp.arange(npad_e, dtype=jnp.int32) % (NPAD - N)
    src2 = jnp.concatenate(
        [edge_index[0], jnp.zeros((npad_e,), jnp.int32)]).reshape(NCHUNKS, CHUNK)
    dst2 = jnp.concatenate(
        [edge_index[1], pad_dst]).reshape(NCHUNKS, CHUNK)
    x_pad = jnp.zeros((NPAD, D), jnp.float32).at[:N].set(x)

    degp = _degree(dst2)
    hs_a, hs_b, dinv = _stage1(x_pad, W1, degp.reshape(NC, NPAD, 1))
    p1a = _segsum_h(hs_a, src2, dst2)
    p1b = _segsum_h(hs_b, src2, dst2)
    h2s = _stage2(p1a, p1b, hs_a, hs_b, dinv, b1.reshape(1, H), W2)
    p2 = _segsum_o(h2s, src2, dst2)
    out2 = _stage3(p2, h2s, dinv, b2.reshape(1, O))
    qrows = _gather_queries(out2, query_node_indices)
    logits = _mlp(qrows, Wm1, bm1.reshape(1, MLP_H), Wm2, bm2.reshape(1, 1))
    return logits[:, 0]


# speedup vs baseline: 1.3017x; 1.0138x over previous
import functools

import jax
import jax.numpy as jnp
from jax import lax
from jax.experimental import pallas as pl
from jax.experimental.pallas import tpu as pltpu
from jax.experimental.pallas import tpu_sc as plsc

N = 10000
E = 320000
D = 128
H = 128
O = 32
Q = 1024
MLP_H = 64

NPAD = 10240
NC = 2
NS = 16
NW = NC * NS
CHUNK = 128
EPAD = 327680
NCHUNKS = EPAD // CHUNK
CH_PER_TILE = NCHUNKS // NW
ROWS_PER_TILE = NPAD // NS

_MESH = dict(core_axis_name="c", subcore_axis_name="s", num_cores=NC,
             num_subcores=NS)
_SC_PARAMS = pltpu.CompilerParams(use_tc_tiling_on_sc=False)


def _fill_zeros_2d(ref, nrows, ncols):
    zv = jnp.zeros((16,), jnp.float32)

    def body(r, _):
        for j in range(ncols // 16):
            ref[r, pl.ds(j * 16, 16)] = zv
        return 0

    lax.fori_loop(0, nrows, body, 0)


def _make_segsum(F, K):
    mesh = plsc.VectorSubcoreMesh(**_MESH)

    @functools.partial(
        pl.kernel,
        out_type=jax.ShapeDtypeStruct((NC, NPAD, F), jnp.float32),
        mesh=mesh,
        compiler_params=_SC_PARAMS,
        scratch_types=[
            pltpu.VMEM((K, CHUNK), jnp.int32),
            pltpu.VMEM((K, CHUNK), jnp.int32),
        ] + [pltpu.VMEM((CHUNK, F), jnp.float32) for _ in range(K)] + [
            pltpu.VMEM_SHARED((NPAD, F), jnp.float32),
            pltpu.SemaphoreType.DMA,
            pltpu.SemaphoreType.DMA,
        ],
    )
    def segsum(hs, src2, dst2, out, src_blk, dst_blk, *rest):
        bufs = rest[:K]
        acc, gsem, ssem = rest[K], rest[K + 1], rest[K + 2]
        c = lax.axis_index("c")
        s = lax.axis_index("s")
        w = c * NS + s
        _fill_zeros_2d(bufs[0], CHUNK, F)
        for t in range(ROWS_PER_TILE // CHUNK):
            pltpu.sync_copy(bufs[0],
                            acc.at[pl.ds(s * ROWS_PER_TILE + t * CHUNK, CHUNK)])
        plsc.subcore_barrier()

        def blk_body(blk, _):
            row0 = (blk * NW + w) * K
            pltpu.sync_copy(src2.at[pl.ds(row0, K), :], src_blk)
            pltpu.sync_copy(dst2.at[pl.ds(row0, K), :], dst_blk)
            g = [pltpu.async_copy(hs.at[src_blk.at[j]], bufs[j], gsem)
                 for j in range(K)]
            sc = []
            for j in range(K):
                g[j].wait()
                sc.append(pltpu.async_copy(bufs[j], acc.at[dst_blk.at[j]],
                                           ssem, add=True))
            for d in sc:
                d.wait()
            return 0

        lax.fori_loop(0, CH_PER_TILE // K, blk_body, 0)
        plsc.subcore_barrier()
        pltpu.sync_copy(acc.at[pl.ds(s * ROWS_PER_TILE, ROWS_PER_TILE)],
                        out.at[c, pl.ds(s * ROWS_PER_TILE, ROWS_PER_TILE), :])

    return segsum


FH = H // 2
_segsum_h = _make_segsum(FH, 10)
_segsum_o = _make_segsum(O, 16)


_KD = 16


@functools.partial(
    pl.kernel,
    out_type=jax.ShapeDtypeStruct((NC, NPAD), jnp.float32),
    mesh=plsc.VectorSubcoreMesh(**_MESH),
    compiler_params=_SC_PARAMS,
    scratch_types=[
        pltpu.VMEM((_KD, CHUNK), jnp.int32),
        pltpu.VMEM((CHUNK,), jnp.float32),
        pltpu.VMEM((ROWS_PER_TILE,), jnp.float32),
        pltpu.VMEM_SHARED((NPAD,), jnp.float32),
        pltpu.SemaphoreType.DMA,
    ],
)
def _degree(dst2, out, dst_blk, ones_v, zer_v, acc, ssem):
    c = lax.axis_index("c")
    s = lax.axis_index("s")
    w = c * NS + s
    one = jnp.full((16,), 1.0, jnp.float32)
    zero = jnp.zeros((16,), jnp.float32)
    for j in range(CHUNK // 16):
        ones_v[pl.ds(j * 16, 16)] = one

    def zbody(r, _):
        zer_v[pl.ds(r * 16, 16)] = zero
        return 0

    lax.fori_loop(0, ROWS_PER_TILE // 16, zbody, 0)
    pltpu.sync_copy(zer_v, acc.at[pl.ds(s * ROWS_PER_TILE, ROWS_PER_TILE)])
    plsc.subcore_barrier()

    def body(blk, _):
        row0 = w * CH_PER_TILE + blk * _KD
        pltpu.sync_copy(dst2.at[pl.ds(row0, _KD), :], dst_blk)
        sc = [pltpu.async_copy(ones_v, acc.at[dst_blk.at[j]], ssem, add=True)
              for j in range(_KD)]
        for d in sc:
            d.wait()
        return 0

    lax.fori_loop(0, CH_PER_TILE // _KD, body, 0)
    plsc.subcore_barrier()
    pltpu.sync_copy(acc.at[pl.ds(s * ROWS_PER_TILE, ROWS_PER_TILE)],
                    out.at[c, pl.ds(s * ROWS_PER_TILE, ROWS_PER_TILE)])


_Q_PER_TILE = Q // (NC * NS)


@functools.partial(
    pl.kernel,
    out_type=jax.ShapeDtypeStruct((Q, O), jnp.float32),
    mesh=plsc.VectorSubcoreMesh(**_MESH),
    compiler_params=_SC_PARAMS,
    scratch_types=[
        pltpu.VMEM((_Q_PER_TILE,), jnp.int32),
        pltpu.VMEM((_Q_PER_TILE, O), jnp.float32),
        pltpu.SemaphoreType.DMA,
    ],
)
def _gather_queries(table, qidx, out, idx_v, rows_v, sem):
    c = lax.axis_index("c")
    s = lax.axis_index("s")
    w = s * NC + c
    base = w * _Q_PER_TILE
    pltpu.sync_copy(qidx.at[pl.ds(base, _Q_PER_TILE)], idx_v)
    pltpu.async_copy(table.at[idx_v], rows_v, sem).wait()
    pltpu.sync_copy(rows_v, out.at[pl.ds(base, _Q_PER_TILE), :])


_NBLK = NPAD // 1024


def _tc_scale1(x_r, w_r, dp_r, hsa_r, hsb_r, dinv_r):
    deg = dp_r[0] + dp_r[1] + 1.0
    dinv = lax.rsqrt(jnp.maximum(deg, 1e-12))
    hs = jnp.dot(x_r[...], w_r[...],
                 preferred_element_type=jnp.float32) * dinv
    hsa_r[...] = hs[:, :FH]
    hsb_r[...] = hs[:, FH:]
    dinv_r[...] = dinv


_stage1 = pl.pallas_call(
    _tc_scale1,
    grid=(_NBLK,),
    in_specs=[
        pl.BlockSpec((1024, D), lambda i: (i, 0)),
        pl.BlockSpec((D, H), lambda i: (0, 0)),
        pl.BlockSpec((NC, 1024, 1), lambda i: (0, i, 0)),
    ],
    out_specs=[
        pl.BlockSpec((1024, FH), lambda i: (i, 0)),
        pl.BlockSpec((1024, FH), lambda i: (i, 0)),
        pl.BlockSpec((1024, 1), lambda i: (i, 0)),
    ],
    out_shape=[
        jax.ShapeDtypeStruct((NPAD, FH), jnp.float32),
        jax.ShapeDtypeStruct((NPAD, FH), jnp.float32),
        jax.ShapeDtypeStruct((NPAD, 1), jnp.float32),
    ],
)


def _tc_combine1(pa_r, pb_r, hsa_r, hsb_r, dinv_r, b1_r, w2_r, h2s_r):
    seg_a = pa_r[0] + pa_r[1] + hsa_r[...]
    seg_b = pb_r[0] + pb_r[1] + hsb_r[...]
    seg = jnp.concatenate([seg_a, seg_b], axis=1)
    out1 = jnp.maximum(dinv_r[...] * seg + b1_r[...], 0.0)
    h2s_r[...] = jnp.dot(out1, w2_r[...],
                         preferred_element_type=jnp.float32) * dinv_r[...]


_stage2 = pl.pallas_call(
    _tc_combine1,
    grid=(_NBLK,),
    in_specs=[
        pl.BlockSpec((NC, 1024, FH), lambda i: (0, i, 0)),
        pl.BlockSpec((NC, 1024, FH), lambda i: (0, i, 0)),
        pl.BlockSpec((1024, FH), lambda i: (i, 0)),
        pl.BlockSpec((1024, FH), lambda i: (i, 0)),
        pl.BlockSpec((1024, 1), lambda i: (i, 0)),
        pl.BlockSpec((1, H), lambda i: (0, 0)),
        pl.BlockSpec((H, O), lambda i: (0, 0)),
    ],
    out_specs=pl.BlockSpec((1024, O), lambda i: (i, 0)),
    out_shape=jax.ShapeDtypeStruct((NPAD, O), jnp.float32),
)


def _tc_combine2(q_r, h2s_r, dinv_r, b2_r, out_r):
    seg = q_r[0] + q_r[1] + h2s_r[...]
    out_r[...] = jnp.maximum(dinv_r[...] * seg + b2_r[...], 0.0)


_stage3 = pl.pallas_call(
    _tc_combine2,
    grid=(_NBLK,),
    in_specs=[
        pl.BlockSpec((NC, 1024, O), lambda i: (0, i, 0)),
        pl.BlockSpec((1024, O), lambda i: (i, 0)),
        pl.BlockSpec((1024, 1), lambda i: (i, 0)),
        pl.BlockSpec((1, O), lambda i: (0, 0)),
    ],
    out_specs=pl.BlockSpec((1024, O), lambda i: (i, 0)),
    out_shape=jax.ShapeDtypeStruct((NPAD, O), jnp.float32),
)


def _tc_mlp(q_r, wm1_r, bm1_r, wm2_r, bm2_r, out_r):
    z = jnp.maximum(jnp.dot(q_r[...], wm1_r[...],
                            preferred_element_type=jnp.float32) + bm1_r[...], 0.0)
    out_r[...] = jnp.dot(z, wm2_r[...],
                         preferred_element_type=jnp.float32) + bm2_r[...]


_mlp = pl.pallas_call(
    _tc_mlp,
    out_shape=jax.ShapeDtypeStruct((Q, 1), jnp.float32),
)


@jax.jit
def kernel(x, edge_index, query_node_indices, W1, b1, W2, b2, Wm1, bm1, Wm2, bm2):
    npad_e = EPAD - E
    pad_dst = N + jnp.arange(npad_e, dtype=jnp.int32) % (NPAD - N)
    src2 = jnp.concatenate(
        [edge_index[0], jnp.zeros((npad_e,), jnp.int32)]).reshape(NCHUNKS, CHUNK)
    dst2 = jnp.concatenate(
        [edge_index[1], pad_dst]).reshape(NCHUNKS, CHUNK)
    x_pad = jnp.zeros((NPAD, D), jnp.float32).at[:N].set(x)

    degp = _degree(dst2)
    hs_a, hs_b, dinv = _stage1(x_pad, W1, degp.reshape(NC, NPAD, 1))
    p1a = _segsum_h(hs_a, src2, dst2)
    p1b = _segsum_h(hs_b, src2, dst2)
    h2s = _stage2(p1a, p1b, hs_a, hs_b, dinv, b1.reshape(1, H), W2)
    p2 = _segsum_o(h2s, src2, dst2)
    out2 = _stage3(p2, h2s, dinv, b2.reshape(1, O))
    qrows = _gather_queries(out2, query_node_indices)
    logits = _mlp(qrows, Wm1, bm1.reshape(1, MLP_H), Wm2, bm2.reshape(1, 1))
    return logits[:, 0]


# speedup vs baseline: 1.6891x; 1.2976x over previous
import functools

import jax
import jax.numpy as jnp
from jax import lax
from jax.experimental import pallas as pl
from jax.experimental.pallas import tpu as pltpu
from jax.experimental.pallas import tpu_sc as plsc

N = 10000
E = 320000
D = 128
H = 128
O = 32
Q = 1024
MLP_H = 64

NPAD = 10240
NC = 2
NS = 16
NW = NC * NS
CHUNK = 128
EPAD = 327680
NCHUNKS = EPAD // CHUNK
CH_PER_TILE = NCHUNKS // NW
ROWS_PER_TILE = NPAD // NS

_MESH = dict(core_axis_name="c", subcore_axis_name="s", num_cores=NC,
             num_subcores=NS)
_SC_PARAMS = pltpu.CompilerParams(use_tc_tiling_on_sc=False)


def _fill_zeros_2d(ref, nrows, ncols):
    zv = jnp.zeros((16,), jnp.float32)

    def body(r, _):
        for j in range(ncols // 16):
            ref[r, pl.ds(j * 16, 16)] = zv
        return 0

    lax.fori_loop(0, nrows, body, 0)


def _make_segsum_merged(F, K):
    mesh = plsc.VectorSubcoreMesh(**_MESH)

    @functools.partial(
        pl.kernel,
        out_type=jax.ShapeDtypeStruct((NC, NPAD, F), jnp.float32),
        mesh=mesh,
        compiler_params=_SC_PARAMS,
        scratch_types=[
            pltpu.VMEM((K, CHUNK), jnp.int32),
            pltpu.VMEM((K, CHUNK), jnp.int32),
        ] + [pltpu.VMEM((CHUNK, F), jnp.float32) for _ in range(K)] + [
            pltpu.VMEM_SHARED((NPAD, F), jnp.float32),
            pltpu.SemaphoreType.DMA,
            pltpu.SemaphoreType.DMA,
        ],
    )
    def segsum(hs2, srcs3, dst2, out, src_blk, dst_blk, *rest):
        bufs = rest[:K]
        acc, gsem, ssem = rest[K], rest[K + 1], rest[K + 2]
        c = lax.axis_index("c")
        s = lax.axis_index("s")
        _fill_zeros_2d(bufs[0], CHUNK, F)
        for t in range(ROWS_PER_TILE // CHUNK):
            pltpu.sync_copy(bufs[0],
                            acc.at[pl.ds(s * ROWS_PER_TILE + t * CHUNK, CHUNK)])
        plsc.subcore_barrier()

        def blk_body(blk, _):
            row0 = (blk * NS + s) * K
            pltpu.sync_copy(srcs3.at[c, pl.ds(row0, K), :], src_blk)
            pltpu.sync_copy(dst2.at[pl.ds(row0, K), :], dst_blk)
            g = [pltpu.async_copy(hs2.at[src_blk.at[j]], bufs[j], gsem)
                 for j in range(K)]
            sc = []
            for j in range(K):
                g[j].wait()
                sc.append(pltpu.async_copy(bufs[j], acc.at[dst_blk.at[j]],
                                           ssem, add=True))
            for d in sc:
                d.wait()
            return 0

        lax.fori_loop(0, NCHUNKS // (NS * K), blk_body, 0)
        plsc.subcore_barrier()
        pltpu.sync_copy(acc.at[pl.ds(s * ROWS_PER_TILE, ROWS_PER_TILE)],
                        out.at[c, pl.ds(s * ROWS_PER_TILE, ROWS_PER_TILE), :])

    return segsum


def _make_segsum(F, K):
    mesh = plsc.VectorSubcoreMesh(**_MESH)

    @functools.partial(
        pl.kernel,
        out_type=jax.ShapeDtypeStruct((NC, NPAD, F), jnp.float32),
        mesh=mesh,
        compiler_params=_SC_PARAMS,
        scratch_types=[
            pltpu.VMEM((K, CHUNK), jnp.int32),
            pltpu.VMEM((K, CHUNK), jnp.int32),
        ] + [pltpu.VMEM((CHUNK, F), jnp.float32) for _ in range(K)] + [
            pltpu.VMEM_SHARED((NPAD, F), jnp.float32),
            pltpu.SemaphoreType.DMA,
            pltpu.SemaphoreType.DMA,
        ],
    )
    def segsum(hs, src2, dst2, out, src_blk, dst_blk, *rest):
        bufs = rest[:K]
        acc, gsem, ssem = rest[K], rest[K + 1], rest[K + 2]
        c = lax.axis_index("c")
        s = lax.axis_index("s")
        w = c * NS + s
        _fill_zeros_2d(bufs[0], CHUNK, F)
        for t in range(ROWS_PER_TILE // CHUNK):
            pltpu.sync_copy(bufs[0],
                            acc.at[pl.ds(s * ROWS_PER_TILE + t * CHUNK, CHUNK)])
        plsc.subcore_barrier()

        def blk_body(blk, _):
            row0 = (blk * NW + w) * K
            pltpu.sync_copy(src2.at[pl.ds(row0, K), :], src_blk)
            pltpu.sync_copy(dst2.at[pl.ds(row0, K), :], dst_blk)
            g = [pltpu.async_copy(hs.at[src_blk.at[j]], bufs[j], gsem)
                 for j in range(K)]
            sc = []
            for j in range(K):
                g[j].wait()
                sc.append(pltpu.async_copy(bufs[j], acc.at[dst_blk.at[j]],
                                           ssem, add=True))
            for d in sc:
                d.wait()
            return 0

        lax.fori_loop(0, CH_PER_TILE // K, blk_body, 0)
        plsc.subcore_barrier()
        pltpu.sync_copy(acc.at[pl.ds(s * ROWS_PER_TILE, ROWS_PER_TILE)],
                        out.at[c, pl.ds(s * ROWS_PER_TILE, ROWS_PER_TILE), :])

    return segsum


FH = H // 2
_segsum_h = _make_segsum_merged(FH, 10)
_segsum_o = _make_segsum(O, 16)


_KD = 16


@functools.partial(
    pl.kernel,
    out_type=jax.ShapeDtypeStruct((NC, NPAD), jnp.float32),
    mesh=plsc.VectorSubcoreMesh(**_MESH),
    compiler_params=_SC_PARAMS,
    scratch_types=[
        pltpu.VMEM((_KD, CHUNK), jnp.int32),
        pltpu.VMEM((CHUNK,), jnp.float32),
        pltpu.VMEM((ROWS_PER_TILE,), jnp.float32),
        pltpu.VMEM_SHARED((NPAD,), jnp.float32),
        pltpu.SemaphoreType.DMA,
    ],
)
def _degree(dst2, out, dst_blk, ones_v, zer_v, acc, ssem):
    c = lax.axis_index("c")
    s = lax.axis_index("s")
    w = c * NS + s
    one = jnp.full((16,), 1.0, jnp.float32)
    zero = jnp.zeros((16,), jnp.float32)
    for j in range(CHUNK // 16):
        ones_v[pl.ds(j * 16, 16)] = one

    def zbody(r, _):
        zer_v[pl.ds(r * 16, 16)] = zero
        return 0

    lax.fori_loop(0, ROWS_PER_TILE // 16, zbody, 0)
    pltpu.sync_copy(zer_v, acc.at[pl.ds(s * ROWS_PER_TILE, ROWS_PER_TILE)])
    plsc.subcore_barrier()

    def body(blk, _):
        row0 = w * CH_PER_TILE + blk * _KD
        pltpu.sync_copy(dst2.at[pl.ds(row0, _KD), :], dst_blk)
        sc = [pltpu.async_copy(ones_v, acc.at[dst_blk.at[j]], ssem, add=True)
              for j in range(_KD)]
        for d in sc:
            d.wait()
        return 0

    lax.fori_loop(0, CH_PER_TILE // _KD, body, 0)
    plsc.subcore_barrier()
    pltpu.sync_copy(acc.at[pl.ds(s * ROWS_PER_TILE, ROWS_PER_TILE)],
                    out.at[c, pl.ds(s * ROWS_PER_TILE, ROWS_PER_TILE)])


_Q_PER_TILE = Q // (NC * NS)


@functools.partial(
    pl.kernel,
    out_type=jax.ShapeDtypeStruct((Q, O), jnp.float32),
    mesh=plsc.VectorSubcoreMesh(**_MESH),
    compiler_params=_SC_PARAMS,
    scratch_types=[
        pltpu.VMEM((_Q_PER_TILE,), jnp.int32),
        pltpu.VMEM((_Q_PER_TILE, O), jnp.float32),
        pltpu.SemaphoreType.DMA,
    ],
)
def _gather_queries(table, qidx, out, idx_v, rows_v, sem):
    c = lax.axis_index("c")
    s = lax.axis_index("s")
    w = s * NC + c
    base = w * _Q_PER_TILE
    pltpu.sync_copy(qidx.at[pl.ds(base, _Q_PER_TILE)], idx_v)
    pltpu.async_copy(table.at[idx_v], rows_v, sem).wait()
    pltpu.sync_copy(rows_v, out.at[pl.ds(base, _Q_PER_TILE), :])


_NBLK = NPAD // 1024


def _tc_scale1(x_r, w_r, dp_r, hs2_r, dinv_r):
    deg = dp_r[0] + dp_r[1] + 1.0
    dinv = lax.rsqrt(jnp.maximum(deg, 1e-12))
    hs = jnp.dot(x_r[...], w_r[...],
                 preferred_element_type=jnp.float32) * dinv
    hs2_r[0] = hs[:, :FH]
    hs2_r[1] = hs[:, FH:]
    dinv_r[...] = dinv


_stage1 = pl.pallas_call(
    _tc_scale1,
    grid=(_NBLK,),
    in_specs=[
        pl.BlockSpec((1024, D), lambda i: (i, 0)),
        pl.BlockSpec((D, H), lambda i: (0, 0)),
        pl.BlockSpec((NC, 1024, 1), lambda i: (0, i, 0)),
    ],
    out_specs=[
        pl.BlockSpec((NC, 1024, FH), lambda i: (0, i, 0)),
        pl.BlockSpec((1024, 1), lambda i: (i, 0)),
    ],
    out_shape=[
        jax.ShapeDtypeStruct((NC, NPAD, FH), jnp.float32),
        jax.ShapeDtypeStruct((NPAD, 1), jnp.float32),
    ],
)


def _tc_combine1(p_r, hs2_r, dinv_r, b1_r, w2_r, h2s_r):
    seg_a = p_r[0] + hs2_r[0]
    seg_b = p_r[1] + hs2_r[1]
    seg = jnp.concatenate([seg_a, seg_b], axis=1)
    out1 = jnp.maximum(dinv_r[...] * seg + b1_r[...], 0.0)
    h2s_r[...] = jnp.dot(out1, w2_r[...],
                         preferred_element_type=jnp.float32) * dinv_r[...]


_stage2 = pl.pallas_call(
    _tc_combine1,
    grid=(_NBLK,),
    in_specs=[
        pl.BlockSpec((NC, 1024, FH), lambda i: (0, i, 0)),
        pl.BlockSpec((NC, 1024, FH), lambda i: (0, i, 0)),
        pl.BlockSpec((1024, 1), lambda i: (i, 0)),
        pl.BlockSpec((1, H), lambda i: (0, 0)),
        pl.BlockSpec((H, O), lambda i: (0, 0)),
    ],
    out_specs=pl.BlockSpec((1024, O), lambda i: (i, 0)),
    out_shape=jax.ShapeDtypeStruct((NPAD, O), jnp.float32),
)


def _tc_combine2(q_r, h2s_r, dinv_r, b2_r, out_r):
    seg = q_r[0] + q_r[1] + h2s_r[...]
    out_r[...] = jnp.maximum(dinv_r[...] * seg + b2_r[...], 0.0)


_stage3 = pl.pallas_call(
    _tc_combine2,
    grid=(_NBLK,),
    in_specs=[
        pl.BlockSpec((NC, 1024, O), lambda i: (0, i, 0)),
        pl.BlockSpec((1024, O), lambda i: (i, 0)),
        pl.BlockSpec((1024, 1), lambda i: (i, 0)),
        pl.BlockSpec((1, O), lambda i: (0, 0)),
    ],
    out_specs=pl.BlockSpec((1024, O), lambda i: (i, 0)),
    out_shape=jax.ShapeDtypeStruct((NPAD, O), jnp.float32),
)


def _tc_mlp(q_r, wm1_r, bm1_r, wm2_r, bm2_r, out_r):
    z = jnp.maximum(jnp.dot(q_r[...], wm1_r[...],
                            preferred_element_type=jnp.float32) + bm1_r[...], 0.0)
    out_r[...] = jnp.dot(z, wm2_r[...],
                         preferred_element_type=jnp.float32) + bm2_r[...]


_mlp = pl.pallas_call(
    _tc_mlp,
    out_shape=jax.ShapeDtypeStruct((Q, 1), jnp.float32),
)


@jax.jit
def kernel(x, edge_index, query_node_indices, W1, b1, W2, b2, Wm1, bm1, Wm2, bm2):
    npad_e = EPAD - E
    pad_dst = N + jnp.arange(npad_e, dtype=jnp.int32) % (NPAD - N)
    src2 = jnp.concatenate(
        [edge_index[0], jnp.zeros((npad_e,), jnp.int32)]).reshape(NCHUNKS, CHUNK)
    dst2 = jnp.concatenate(
        [edge_index[1], pad_dst]).reshape(NCHUNKS, CHUNK)
    x_pad = jnp.zeros((NPAD, D), jnp.float32).at[:N].set(x)

    srcs3 = jnp.stack([src2, src2 + NPAD])
    degp = _degree(dst2)
    hs2, dinv = _stage1(x_pad, W1, degp.reshape(NC, NPAD, 1))
    p1 = _segsum_h(hs2.reshape(NC * NPAD, FH), srcs3, dst2)
    h2s = _stage2(p1, hs2, dinv, b1.reshape(1, H), W2)
    p2 = _segsum_o(h2s, src2, dst2)
    out2 = _stage3(p2, h2s, dinv, b2.reshape(1, O))
    qrows = _gather_queries(out2, query_node_indices)
    logits = _mlp(qrows, Wm1, bm1.reshape(1, MLP_H), Wm2, bm2.reshape(1, 1))
    return logits[:, 0]


# speedup vs baseline: 1.7460x; 1.0337x over previous
import functools

import jax
import jax.numpy as jnp
from jax import lax
from jax.experimental import pallas as pl
from jax.experimental.pallas import tpu as pltpu
from jax.experimental.pallas import tpu_sc as plsc

N = 10000
E = 320000
D = 128
H = 128
O = 32
Q = 1024
MLP_H = 64

NPAD = 10240
NC = 2
NS = 16
NW = NC * NS
CHUNK = 128
EPAD = 327680
NCHUNKS = EPAD // CHUNK
CH_PER_TILE = NCHUNKS // NW
ROWS_PER_TILE = NPAD // NS

_MESH = dict(core_axis_name="c", subcore_axis_name="s", num_cores=NC,
             num_subcores=NS)
_SC_PARAMS = pltpu.CompilerParams(use_tc_tiling_on_sc=False)


def _fill_zeros_2d(ref, nrows, ncols):
    zv = jnp.zeros((16,), jnp.float32)

    def body(r, _):
        for j in range(ncols // 16):
            ref[r, pl.ds(j * 16, 16)] = zv
        return 0

    lax.fori_loop(0, nrows, body, 0)


def _make_segsum_merged(F, K):
    mesh = plsc.VectorSubcoreMesh(**_MESH)

    @functools.partial(
        pl.kernel,
        out_type=jax.ShapeDtypeStruct((NC, NPAD, F), jnp.float32),
        mesh=mesh,
        compiler_params=_SC_PARAMS,
        scratch_types=[
            pltpu.VMEM((K, CHUNK), jnp.int32),
            pltpu.VMEM((K, CHUNK), jnp.int32),
        ] + [pltpu.VMEM((CHUNK, F), jnp.float32) for _ in range(K)] + [
            pltpu.VMEM_SHARED((NPAD, F), jnp.float32),
            pltpu.SemaphoreType.DMA,
            pltpu.SemaphoreType.DMA,
        ],
    )
    def segsum(hs2, srcs3, dst2, out, src_blk, dst_blk, *rest):
        bufs = rest[:K]
        acc, gsem, ssem = rest[K], rest[K + 1], rest[K + 2]
        c = lax.axis_index("c")
        s = lax.axis_index("s")
        _fill_zeros_2d(bufs[0], CHUNK, F)
        for t in range(ROWS_PER_TILE // CHUNK):
            pltpu.sync_copy(bufs[0],
                            acc.at[pl.ds(s * ROWS_PER_TILE + t * CHUNK, CHUNK)])
        plsc.subcore_barrier()

        def blk_body(blk, _):
            row0 = (blk * NS + s) * K
            pltpu.sync_copy(srcs3.at[c, pl.ds(row0, K), :], src_blk)
            pltpu.sync_copy(dst2.at[pl.ds(row0, K), :], dst_blk)
            g = [pltpu.async_copy(hs2.at[src_blk.at[j]], bufs[j], gsem)
                 for j in range(K)]
            sc = []
            for j in range(K):
                g[j].wait()
                sc.append(pltpu.async_copy(bufs[j], acc.at[dst_blk.at[j]],
                                           ssem, add=True))
            for d in sc:
                d.wait()
            return 0

        lax.fori_loop(0, NCHUNKS // (NS * K), blk_body, 0)
        plsc.subcore_barrier()
        pltpu.sync_copy(acc.at[pl.ds(s * ROWS_PER_TILE, ROWS_PER_TILE)],
                        out.at[c, pl.ds(s * ROWS_PER_TILE, ROWS_PER_TILE), :])

    return segsum


def _make_segsum(F, K):
    mesh = plsc.VectorSubcoreMesh(**_MESH)

    @functools.partial(
        pl.kernel,
        out_type=jax.ShapeDtypeStruct((NC, NPAD, F), jnp.float32),
        mesh=mesh,
        compiler_params=_SC_PARAMS,
        scratch_types=[
            pltpu.VMEM((K, CHUNK), jnp.int32),
            pltpu.VMEM((K, CHUNK), jnp.int32),
        ] + [pltpu.VMEM((CHUNK, F), jnp.float32) for _ in range(K)] + [
            pltpu.VMEM_SHARED((NPAD, F), jnp.float32),
            pltpu.SemaphoreType.DMA,
            pltpu.SemaphoreType.DMA,
        ],
    )
    def segsum(hs, src2, dst2, out, src_blk, dst_blk, *rest):
        bufs = rest[:K]
        acc, gsem, ssem = rest[K], rest[K + 1], rest[K + 2]
        c = lax.axis_index("c")
        s = lax.axis_index("s")
        w = c * NS + s
        _fill_zeros_2d(bufs[0], CHUNK, F)
        for t in range(ROWS_PER_TILE // CHUNK):
            pltpu.sync_copy(bufs[0],
                            acc.at[pl.ds(s * ROWS_PER_TILE + t * CHUNK, CHUNK)])
        plsc.subcore_barrier()

        def blk_body(blk, _):
            row0 = (blk * NW + w) * K
            pltpu.sync_copy(src2.at[pl.ds(row0, K), :], src_blk)
            pltpu.sync_copy(dst2.at[pl.ds(row0, K), :], dst_blk)
            g = [pltpu.async_copy(hs.at[src_blk.at[j]], bufs[j], gsem)
                 for j in range(K)]
            sc = []
            for j in range(K):
                g[j].wait()
                sc.append(pltpu.async_copy(bufs[j], acc.at[dst_blk.at[j]],
                                           ssem, add=True))
            for d in sc:
                d.wait()
            return 0

        lax.fori_loop(0, CH_PER_TILE // K, blk_body, 0)
        plsc.subcore_barrier()
        pltpu.sync_copy(acc.at[pl.ds(s * ROWS_PER_TILE, ROWS_PER_TILE)],
                        out.at[c, pl.ds(s * ROWS_PER_TILE, ROWS_PER_TILE), :])

    return segsum


FH = H // 2
FO = O // 2
_segsum_h = _make_segsum_merged(FH, 10)
_segsum_o = _make_segsum_merged(FO, 16)


_KD = 16


@functools.partial(
    pl.kernel,
    out_type=jax.ShapeDtypeStruct((NC, NPAD), jnp.float32),
    mesh=plsc.VectorSubcoreMesh(**_MESH),
    compiler_params=_SC_PARAMS,
    scratch_types=[
        pltpu.VMEM((_KD, CHUNK), jnp.int32),
        pltpu.VMEM((CHUNK,), jnp.float32),
        pltpu.VMEM((ROWS_PER_TILE,), jnp.float32),
        pltpu.VMEM_SHARED((NPAD,), jnp.float32),
        pltpu.SemaphoreType.DMA,
    ],
)
def _degree(dst2, out, dst_blk, ones_v, zer_v, acc, ssem):
    c = lax.axis_index("c")
    s = lax.axis_index("s")
    w = c * NS + s
    one = jnp.full((16,), 1.0, jnp.float32)
    zero = jnp.zeros((16,), jnp.float32)
    for j in range(CHUNK // 16):
        ones_v[pl.ds(j * 16, 16)] = one

    def zbody(r, _):
        zer_v[pl.ds(r * 16, 16)] = zero
        return 0

    lax.fori_loop(0, ROWS_PER_TILE // 16, zbody, 0)
    pltpu.sync_copy(zer_v, acc.at[pl.ds(s * ROWS_PER_TILE, ROWS_PER_TILE)])
    plsc.subcore_barrier()

    def body(blk, _):
        row0 = w * CH_PER_TILE + blk * _KD
        pltpu.sync_copy(dst2.at[pl.ds(row0, _KD), :], dst_blk)
        sc = [pltpu.async_copy(ones_v, acc.at[dst_blk.at[j]], ssem, add=True)
              for j in range(_KD)]
        for d in sc:
            d.wait()
        return 0

    lax.fori_loop(0, CH_PER_TILE // _KD, body, 0)
    plsc.subcore_barrier()
    pltpu.sync_copy(acc.at[pl.ds(s * ROWS_PER_TILE, ROWS_PER_TILE)],
                    out.at[c, pl.ds(s * ROWS_PER_TILE, ROWS_PER_TILE)])


_Q_PER_TILE = Q // (NC * NS)


@functools.partial(
    pl.kernel,
    out_type=jax.ShapeDtypeStruct((Q, O), jnp.float32),
    mesh=plsc.VectorSubcoreMesh(**_MESH),
    compiler_params=_SC_PARAMS,
    scratch_types=[
        pltpu.VMEM((_Q_PER_TILE,), jnp.int32),
        pltpu.VMEM((_Q_PER_TILE, O), jnp.float32),
        pltpu.SemaphoreType.DMA,
    ],
)
def _gather_queries(table, qidx, out, idx_v, rows_v, sem):
    c = lax.axis_index("c")
    s = lax.axis_index("s")
    w = s * NC + c
    base = w * _Q_PER_TILE
    pltpu.sync_copy(qidx.at[pl.ds(base, _Q_PER_TILE)], idx_v)
    pltpu.async_copy(table.at[idx_v], rows_v, sem).wait()
    pltpu.sync_copy(rows_v, out.at[pl.ds(base, _Q_PER_TILE), :])


_NBLK = NPAD // 1024


def _tc_scale1(x_r, w_r, dp_r, hs2_r, dinv_r):
    deg = dp_r[0] + dp_r[1] + 1.0
    dinv = lax.rsqrt(jnp.maximum(deg, 1e-12))
    hs = jnp.dot(x_r[...], w_r[...],
                 preferred_element_type=jnp.float32) * dinv
    hs2_r[0] = hs[:, :FH]
    hs2_r[1] = hs[:, FH:]
    dinv_r[...] = dinv


_stage1 = pl.pallas_call(
    _tc_scale1,
    grid=(_NBLK,),
    in_specs=[
        pl.BlockSpec((1024, D), lambda i: (i, 0)),
        pl.BlockSpec((D, H), lambda i: (0, 0)),
        pl.BlockSpec((NC, 1024, 1), lambda i: (0, i, 0)),
    ],
    out_specs=[
        pl.BlockSpec((NC, 1024, FH), lambda i: (0, i, 0)),
        pl.BlockSpec((1024, 1), lambda i: (i, 0)),
    ],
    out_shape=[
        jax.ShapeDtypeStruct((NC, NPAD, FH), jnp.float32),
        jax.ShapeDtypeStruct((NPAD, 1), jnp.float32),
    ],
)


def _tc_combine1(p_r, hs2_r, dinv_r, b1_r, w2_r, h2s2_r):
    seg_a = p_r[0] + hs2_r[0]
    seg_b = p_r[1] + hs2_r[1]
    seg = jnp.concatenate([seg_a, seg_b], axis=1)
    out1 = jnp.maximum(dinv_r[...] * seg + b1_r[...], 0.0)
    h2s = jnp.dot(out1, w2_r[...],
                  preferred_element_type=jnp.float32) * dinv_r[...]
    h2s2_r[0] = h2s[:, :FO]
    h2s2_r[1] = h2s[:, FO:]


_stage2 = pl.pallas_call(
    _tc_combine1,
    grid=(_NBLK,),
    in_specs=[
        pl.BlockSpec((NC, 1024, FH), lambda i: (0, i, 0)),
        pl.BlockSpec((NC, 1024, FH), lambda i: (0, i, 0)),
        pl.BlockSpec((1024, 1), lambda i: (i, 0)),
        pl.BlockSpec((1, H), lambda i: (0, 0)),
        pl.BlockSpec((H, O), lambda i: (0, 0)),
    ],
    out_specs=pl.BlockSpec((NC, 1024, FO), lambda i: (0, i, 0)),
    out_shape=jax.ShapeDtypeStruct((NC, NPAD, FO), jnp.float32),
)


def _tc_combine2(q_r, h2s2_r, dinv_r, b2_r, out_r):
    seg_a = q_r[0] + h2s2_r[0]
    seg_b = q_r[1] + h2s2_r[1]
    seg = jnp.concatenate([seg_a, seg_b], axis=1)
    out_r[...] = jnp.maximum(dinv_r[...] * seg + b2_r[...], 0.0)


_stage3 = pl.pallas_call(
    _tc_combine2,
    grid=(_NBLK,),
    in_specs=[
        pl.BlockSpec((NC, 1024, FO), lambda i: (0, i, 0)),
        pl.BlockSpec((NC, 1024, FO), lambda i: (0, i, 0)),
        pl.BlockSpec((1024, 1), lambda i: (i, 0)),
        pl.BlockSpec((1, O), lambda i: (0, 0)),
    ],
    out_specs=pl.BlockSpec((1024, O), lambda i: (i, 0)),
    out_shape=jax.ShapeDtypeStruct((NPAD, O), jnp.float32),
)


def _tc_mlp(q_r, wm1_r, bm1_r, wm2_r, bm2_r, out_r):
    z = jnp.maximum(jnp.dot(q_r[...], wm1_r[...],
                            preferred_element_type=jnp.float32) + bm1_r[...], 0.0)
    out_r[...] = jnp.dot(z, wm2_r[...],
                         preferred_element_type=jnp.float32) + bm2_r[...]


_mlp = pl.pallas_call(
    _tc_mlp,
    out_shape=jax.ShapeDtypeStruct((Q, 1), jnp.float32),
)


@jax.jit
def kernel(x, edge_index, query_node_indices, W1, b1, W2, b2, Wm1, bm1, Wm2, bm2):
    npad_e = EPAD - E
    pad_dst = N + jnp.arange(npad_e, dtype=jnp.int32) % (NPAD - N)
    src2 = jnp.concatenate(
        [edge_index[0], jnp.zeros((npad_e,), jnp.int32)]).reshape(NCHUNKS, CHUNK)
    dst2 = jnp.concatenate(
        [edge_index[1], pad_dst]).reshape(NCHUNKS, CHUNK)
    x_pad = jnp.zeros((NPAD, D), jnp.float32).at[:N].set(x)

    srcs3 = jnp.stack([src2, src2 + NPAD])
    degp = _degree(dst2)
    hs2, dinv = _stage1(x_pad, W1, degp.reshape(NC, NPAD, 1))
    p1 = _segsum_h(hs2.reshape(NC * NPAD, FH), srcs3, dst2)
    h2s2 = _stage2(p1, hs2, dinv, b1.reshape(1, H), W2)
    p2 = _segsum_o(h2s2.reshape(NC * NPAD, FO), srcs3, dst2)
    out2 = _stage3(p2, h2s2, dinv, b2.reshape(1, O))
    qrows = _gather_queries(out2, query_node_indices)
    logits = _mlp(qrows, Wm1, bm1.reshape(1, MLP_H), Wm2, bm2.reshape(1, 1))
    return logits[:, 0]
